# Initial kernel scaffold; baseline (speedup 1.0000x reference)
import jax
import jax.numpy as jnp
from jax.experimental import pallas as pl


def kernel(V, sqrtlam, x, y, t):
    raise NotImplementedError("write your pallas kernel here")


# speedup vs baseline: 3.0316x; 3.0316x over previous
import jax
import jax.numpy as jnp
from jax import lax
from jax.experimental import pallas as pl
from jax.experimental.pallas import tpu as pltpu

EPS = 1e-5
NCORES = 2
NCHUNK = 8


def _fused_kernel(t_ref, sl_ref, v_ref, xy_ref, p_ref, vs_ref, acc_ref, w_ref):
    s = pl.program_id(1)
    half = NCHUNK
    r_rows = v_ref.shape[0]
    cf = w_ref.shape[-1]

    @pl.when(s < half)
    def _phase1():
        vblk = v_ref[...]
        vs_ref[pl.ds(s * r_rows, r_rows), :] = vblk
        z = lax.dot_general(
            vblk, xy_ref[...],
            dimension_numbers=(((0,), (0,)), ((), ())),
            preferred_element_type=jnp.float32)

        @pl.when(s == 0)
        def _():
            acc_ref[...] = z

        @pl.when(s > 0)
        def _():
            acc_ref[...] += z

        @pl.when(s == half - 1)
        def _():
            t = t_ref[0, 0]
            sv = sl_ref[...]
            cos_d = jnp.cos(t * sv)
            small = jnp.abs(sv) < EPS
            sinc_d = jnp.where(small, t,
                               jnp.sin(t * sv) / jnp.where(small, 1.0, sv))
            z_all = acc_ref[...]
            w_ref[...] = cos_d * z_all[:, :cf] + sinc_d * z_all[:, cf:]

    @pl.when(s >= half)
    def _phase2():
        r = s - half
        vblk = vs_ref[pl.ds(r * r_rows, r_rows), :]
        p_ref[0, :, :] = lax.dot_general(
            vblk, w_ref[...],
            dimension_numbers=(((1,), (0,)), ((), ())),
            preferred_element_type=jnp.float32)


def _combine_kernel(p_ref, o_ref):
    o_ref[...] = p_ref[0] + p_ref[1]


def kernel(V, sqrtlam, x, y, t):
    N, C, F = x.shape
    CF = C * F
    SH = N // NCORES
    R = N // NCHUNK

    xy = jnp.concatenate(
        [x.reshape(N, CF), y.reshape(N, CF)], axis=-1).astype(jnp.float32)
    sl = sqrtlam.reshape(N, 1).astype(jnp.float32)
    t_arr = jnp.full((1, 1), t, dtype=jnp.float32)
    Vf = V.astype(jnp.float32)

    grid = (NCORES, 2 * NCHUNK)
    last = NCHUNK - 1

    p = pl.pallas_call(
        _fused_kernel,
        out_shape=jax.ShapeDtypeStruct((NCORES, N, CF), jnp.float32),
        grid=grid,
        in_specs=[
            pl.BlockSpec(memory_space=pltpu.MemorySpace.SMEM),
            pl.BlockSpec((SH, 1), lambda c, s: (c, 0)),
            pl.BlockSpec((R, SH), lambda c, s: (jnp.minimum(s, last), c)),
            pl.BlockSpec((R, 2 * CF), lambda c, s: (jnp.minimum(s, last), 0)),
        ],
        out_specs=pl.BlockSpec(
            (1, R, CF), lambda c, s: (c, jnp.maximum(s - NCHUNK, 0), 0)),
        scratch_shapes=[
            pltpu.VMEM((N, SH), jnp.float32),
            pltpu.VMEM((SH, 2 * CF), jnp.float32),
            pltpu.VMEM((SH, CF), jnp.float32),
        ],
        compiler_params=pltpu.CompilerParams(
            dimension_semantics=("parallel", "arbitrary"),
            vmem_limit_bytes=100 * 1024 * 1024),
        cost_estimate=pl.CostEstimate(
            flops=2 * N * N * 3 * CF,
            transcendentals=2 * N,
            bytes_accessed=4 * (N * N + 3 * N * CF + NCORES * N * CF)),
    )(t_arr, sl, Vf, xy)

    out = pl.pallas_call(
        _combine_kernel,
        out_shape=jax.ShapeDtypeStruct((N, CF), jnp.float32),
        grid=(NCHUNK,),
        in_specs=[pl.BlockSpec((NCORES, R, CF), lambda s: (0, s, 0))],
        out_specs=pl.BlockSpec((R, CF), lambda s: (s, 0)),
        compiler_params=pltpu.CompilerParams(
            dimension_semantics=("parallel",)),
    )(p)

    return out.reshape(N, C, F)


# speedup vs baseline: 3.3954x; 1.1200x over previous
import jax
import jax.numpy as jnp
from jax import lax
from jax.experimental import pallas as pl
from jax.experimental.pallas import tpu as pltpu

EPS = 1e-5
NCHUNK = 16


def _wave_kernel(cos_ref, sinc_ref, v_ref, x_ref, y_ref, o_ref,
                 vs_ref, acc_ref, w_ref):
    s = pl.program_id(0)
    r_rows = v_ref.shape[0]
    cf = w_ref.shape[-1]

    @pl.when(s < NCHUNK)
    def _project():
        vblk = v_ref[...].astype(jnp.bfloat16)
        vs_ref[pl.ds(s * r_rows, r_rows), :] = vblk
        zx = lax.dot_general(
            vblk, x_ref[...],
            dimension_numbers=(((0,), (0,)), ((), ())),
            preferred_element_type=jnp.float32)
        zy = lax.dot_general(
            vblk, y_ref[...],
            dimension_numbers=(((0,), (0,)), ((), ())),
            preferred_element_type=jnp.float32)

        @pl.when(s == 0)
        def _():
            acc_ref[:, :cf] = zx
            acc_ref[:, cf:] = zy

        @pl.when(s > 0)
        def _():
            acc_ref[:, :cf] += zx
            acc_ref[:, cf:] += zy

        @pl.when(s == NCHUNK - 1)
        def _():
            w_ref[...] = (cos_ref[...] * acc_ref[:, :cf] +
                          sinc_ref[...] * acc_ref[:, cf:]).astype(jnp.bfloat16)

    @pl.when(s >= NCHUNK)
    def _reconstruct():
        r = s - NCHUNK
        vblk = vs_ref[pl.ds(r * r_rows, r_rows), :]
        o_ref[...] = lax.dot_general(
            vblk, w_ref[...],
            dimension_numbers=(((1,), (0,)), ((), ())),
            preferred_element_type=jnp.float32)


def kernel(V, sqrtlam, x, y, t):
    N, C, F = x.shape
    CF = C * F
    R = N // NCHUNK

    xf = x.reshape(N, CF).astype(jnp.float32)
    yf = y.reshape(N, CF).astype(jnp.float32)
    sl = sqrtlam.astype(jnp.float32)
    tf = jnp.asarray(t, dtype=jnp.float32)
    ts = tf * sl
    cos_d = jnp.cos(ts).reshape(N, 1)
    small = jnp.abs(sl) < EPS
    sinc_d = jnp.where(small, tf,
                       jnp.sin(ts) / jnp.where(small, 1.0, sl)).reshape(N, 1)
    Vf = V.astype(jnp.float32)

    last = NCHUNK - 1
    out = pl.pallas_call(
        _wave_kernel,
        out_shape=jax.ShapeDtypeStruct((N, CF), jnp.float32),
        grid=(2 * NCHUNK,),
        in_specs=[
            pl.BlockSpec((N, 1), lambda s: (0, 0)),
            pl.BlockSpec((N, 1), lambda s: (0, 0)),
            pl.BlockSpec((R, N), lambda s: (jnp.minimum(s, last), 0)),
            pl.BlockSpec((R, CF), lambda s: (jnp.minimum(s, last), 0)),
            pl.BlockSpec((R, CF), lambda s: (jnp.minimum(s, last), 0)),
        ],
        out_specs=pl.BlockSpec(
            (R, CF), lambda s: (jnp.maximum(s - NCHUNK, 0), 0)),
        scratch_shapes=[
            pltpu.VMEM((N, N), jnp.bfloat16),
            pltpu.VMEM((N, 2 * CF), jnp.float32),
            pltpu.VMEM((N, CF), jnp.bfloat16),
        ],
        compiler_params=pltpu.CompilerParams(
            dimension_semantics=("arbitrary",),
            vmem_limit_bytes=100 * 1024 * 1024),
        cost_estimate=pl.CostEstimate(
            flops=2 * N * N * 3 * CF,
            transcendentals=0,
            bytes_accessed=4 * (N * N + 3 * N * CF) + 2 * N),
    )(cos_d, sinc_d, Vf, xf, yf)

    return out.reshape(N, C, F)


# speedup vs baseline: 4.7878x; 1.4101x over previous
import jax
import jax.numpy as jnp
from jax import lax
from jax.experimental import pallas as pl
from jax.experimental.pallas import tpu as pltpu

EPS = 1e-5
NCHUNK = 16


def _wave_kernel(cos_ref, sinc_ref, xyt_ref, v_ref, o_ref,
                 vs_ref, zt_ref, wt_ref):
    s = pl.program_id(0)
    cc = v_ref.shape[-1]
    cf = wt_ref.shape[0]

    @pl.when(s < NCHUNK)
    def _project():
        vs_ref[:, pl.ds(s * cc, cc)] = v_ref[...].astype(jnp.bfloat16)
        zt_ref[:, pl.ds(s * cc, cc)] = lax.dot_general(
            xyt_ref[...], vs_ref[:, pl.ds(s * cc, cc)],
            dimension_numbers=(((1,), (0,)), ((), ())),
            preferred_element_type=jnp.float32)

    @pl.when(s == NCHUNK)
    def _reconstruct():
        zt = zt_ref[...]
        wt_ref[...] = (cos_ref[...] * zt[:cf, :] +
                       sinc_ref[...] * zt[cf:, :]).astype(jnp.bfloat16)
        o_ref[...] = lax.dot_general(
            vs_ref[...], wt_ref[...],
            dimension_numbers=(((1,), (1,)), ((), ())),
            preferred_element_type=jnp.float32)


def kernel(V, sqrtlam, x, y, t):
    N, C, F = x.shape
    CF = C * F
    CC = N // NCHUNK

    xf = x.reshape(N, CF)
    yf = y.reshape(N, CF)
    xyt = jnp.concatenate([xf, yf], axis=-1).T.astype(jnp.bfloat16)
    sl = sqrtlam.astype(jnp.float32)
    tf = jnp.asarray(t, dtype=jnp.float32)
    ts = tf * sl
    cos_d = jnp.cos(ts).reshape(1, N)
    small = jnp.abs(sl) < EPS
    sinc_d = jnp.where(small, tf,
                       jnp.sin(ts) / jnp.where(small, 1.0, sl)).reshape(1, N)
    Vf = V.astype(jnp.float32)

    last = NCHUNK - 1
    out = pl.pallas_call(
        _wave_kernel,
        out_shape=jax.ShapeDtypeStruct((N, CF), jnp.float32),
        grid=(NCHUNK + 1,),
        in_specs=[
            pl.BlockSpec((1, N), lambda s: (0, 0)),
            pl.BlockSpec((1, N), lambda s: (0, 0)),
            pl.BlockSpec((2 * CF, N), lambda s: (0, 0)),
            pl.BlockSpec((N, CC), lambda s: (0, jnp.minimum(s, last))),
        ],
        out_specs=pl.BlockSpec((N, CF), lambda s: (0, 0)),
        scratch_shapes=[
            pltpu.VMEM((N, N), jnp.bfloat16),
            pltpu.VMEM((2 * CF, N), jnp.float32),
            pltpu.VMEM((CF, N), jnp.bfloat16),
        ],
        compiler_params=pltpu.CompilerParams(
            dimension_semantics=("arbitrary",),
            vmem_limit_bytes=110 * 1024 * 1024),
        cost_estimate=pl.CostEstimate(
            flops=2 * N * N * 3 * CF,
            transcendentals=0,
            bytes_accessed=4 * (N * N + 3 * N * CF) + 2 * N),
    )(cos_d, sinc_d, xyt, Vf)

    return out.reshape(N, C, F)


# speedup vs baseline: 4.9473x; 1.0333x over previous
import jax
import jax.numpy as jnp
from jax import lax
from jax.experimental import pallas as pl
from jax.experimental.pallas import tpu as pltpu

EPS = 1e-5
NCHUNK = 16
ROWCHUNK = 8


def _wave_kernel(t_ref, sl_ref, xyt_ref, v_ref, o_ref,
                 vs_ref, zt_ref, wt_ref):
    s = pl.program_id(0)
    cc = v_ref.shape[-1]
    cf = wt_ref.shape[0]

    @pl.when(s < NCHUNK)
    def _project():
        vs_ref[:, pl.ds(s * cc, cc)] = v_ref[...].astype(jnp.bfloat16)
        zt_ref[:, pl.ds(s * cc, cc)] = lax.dot_general(
            xyt_ref[...], vs_ref[:, pl.ds(s * cc, cc)],
            dimension_numbers=(((1,), (0,)), ((), ())),
            preferred_element_type=jnp.float32)

    @pl.when(s == NCHUNK)
    def _reconstruct():
        t = t_ref[0, 0]
        sv = sl_ref[...]
        cos_d = jnp.cos(t * sv)
        small = jnp.abs(sv) < EPS
        sinc_d = jnp.where(small, t,
                           jnp.sin(t * sv) / jnp.where(small, 1.0, sv))
        zt = zt_ref[...]
        wt_ref[...] = (cos_d * zt[:cf, :] +
                       sinc_d * zt[cf:, :]).astype(jnp.bfloat16)
        n = vs_ref.shape[0]
        rr = n // ROWCHUNK
        for rb in range(ROWCHUNK):
            o_ref[rb * rr:(rb + 1) * rr, :] = lax.dot_general(
                vs_ref[rb * rr:(rb + 1) * rr, :], wt_ref[...],
                dimension_numbers=(((1,), (1,)), ((), ())),
                preferred_element_type=jnp.float32)


def kernel(V, sqrtlam, x, y, t):
    N, C, F = x.shape
    CF = C * F
    CC = N // NCHUNK

    xf = x.reshape(N, CF)
    yf = y.reshape(N, CF)
    xyt = jnp.concatenate([xf, yf], axis=-1).T.astype(jnp.bfloat16)
    sl = sqrtlam.reshape(1, N).astype(jnp.float32)
    t_arr = jnp.full((1, 1), t, dtype=jnp.float32)
    Vf = V.astype(jnp.float32)

    last = NCHUNK - 1
    out = pl.pallas_call(
        _wave_kernel,
        out_shape=jax.ShapeDtypeStruct((N, CF), jnp.float32),
        grid=(NCHUNK + 1,),
        in_specs=[
            pl.BlockSpec(memory_space=pltpu.MemorySpace.SMEM),
            pl.BlockSpec((1, N), lambda s: (0, 0)),
            pl.BlockSpec((2 * CF, N), lambda s: (0, 0)),
            pl.BlockSpec((N, CC), lambda s: (0, jnp.minimum(s, last))),
        ],
        out_specs=pl.BlockSpec((N, CF), lambda s: (0, 0)),
        scratch_shapes=[
            pltpu.VMEM((N, N), jnp.bfloat16),
            pltpu.VMEM((2 * CF, N), jnp.float32),
            pltpu.VMEM((CF, N), jnp.bfloat16),
        ],
        compiler_params=pltpu.CompilerParams(
            dimension_semantics=("arbitrary",),
            vmem_limit_bytes=110 * 1024 * 1024),
        cost_estimate=pl.CostEstimate(
            flops=2 * N * N * 3 * CF,
            transcendentals=2 * N,
            bytes_accessed=4 * (N * N + 3 * N * CF) + 2 * N),
    )(t_arr, sl, xyt, Vf)

    return out.reshape(N, C, F)


# speedup vs baseline: 5.2516x; 1.0615x over previous
import jax
import jax.numpy as jnp
from jax import lax
from jax.experimental import pallas as pl
from jax.experimental.pallas import tpu as pltpu

EPS = 1e-5
NCHUNK = 8
ROWCHUNK = 8


def _wave_kernel(t_ref, sl_ref, xyt_ref, v_ref, o_ref, vs_ref, wt_ref):
    s = pl.program_id(0)
    cc = v_ref.shape[-1]
    cf = wt_ref.shape[0]

    @pl.when(s < NCHUNK)
    def _project():
        vs_ref[:, pl.ds(s * cc, cc)] = v_ref[...].astype(jnp.bfloat16)
        zt = lax.dot_general(
            xyt_ref[...], vs_ref[:, pl.ds(s * cc, cc)],
            dimension_numbers=(((1,), (0,)), ((), ())),
            preferred_element_type=jnp.float32)
        t = t_ref[0, 0]
        sv = sl_ref[0]
        cos_d = jnp.cos(t * sv)
        small = jnp.abs(sv) < EPS
        sinc_d = jnp.where(small, t,
                           jnp.sin(t * sv) / jnp.where(small, 1.0, sv))
        wt_ref[:, pl.ds(s * cc, cc)] = (
            cos_d * zt[:cf, :] + sinc_d * zt[cf:, :]).astype(jnp.bfloat16)

    @pl.when(s == NCHUNK)
    def _reconstruct():
        n = vs_ref.shape[0]
        rr = n // ROWCHUNK
        for rb in range(ROWCHUNK):
            o_ref[rb * rr:(rb + 1) * rr, :] = lax.dot_general(
                vs_ref[rb * rr:(rb + 1) * rr, :], wt_ref[...],
                dimension_numbers=(((1,), (1,)), ((), ())),
                preferred_element_type=jnp.float32)


def kernel(V, sqrtlam, x, y, t):
    N, C, F = x.shape
    CF = C * F
    CC = N // NCHUNK

    xf = x.reshape(N, CF)
    yf = y.reshape(N, CF)
    xyt = jnp.concatenate([xf, yf], axis=-1).T.astype(jnp.bfloat16)
    sl = sqrtlam.reshape(NCHUNK, 1, CC).astype(jnp.float32)
    t_arr = jnp.full((1, 1), t, dtype=jnp.float32)
    Vf = V.astype(jnp.float32)

    last = NCHUNK - 1
    out = pl.pallas_call(
        _wave_kernel,
        out_shape=jax.ShapeDtypeStruct((N, CF), jnp.float32),
        grid=(NCHUNK + 1,),
        in_specs=[
            pl.BlockSpec(memory_space=pltpu.MemorySpace.SMEM),
            pl.BlockSpec((1, 1, CC), lambda s: (jnp.minimum(s, last), 0, 0)),
            pl.BlockSpec((2 * CF, N), lambda s: (0, 0)),
            pl.BlockSpec((N, CC), lambda s: (0, jnp.minimum(s, last))),
        ],
        out_specs=pl.BlockSpec((N, CF), lambda s: (0, 0)),
        scratch_shapes=[
            pltpu.VMEM((N, N), jnp.bfloat16),
            pltpu.VMEM((CF, N), jnp.bfloat16),
        ],
        compiler_params=pltpu.CompilerParams(
            dimension_semantics=("arbitrary",),
            vmem_limit_bytes=110 * 1024 * 1024),
        cost_estimate=pl.CostEstimate(
            flops=2 * N * N * 3 * CF,
            transcendentals=2 * N,
            bytes_accessed=4 * (N * N + 3 * N * CF) + 2 * N),
    )(t_arr, sl, xyt, Vf)

    return out.reshape(N, C, F)
